# baseline (speedup 1.0000x reference)
import functools

import jax
import jax.numpy as jnp
from jax import lax
from jax.experimental import pallas as pl
from jax.experimental.pallas import tpu as pltpu
from jax.experimental.pallas import tpu_sc as plsc

MAX_LEN = 200
D_MODEL = 64
BATCH = 4096

NUM_CORES = 2
NUM_SUBCORES = 16
NUM_WORKERS = NUM_CORES * NUM_SUBCORES
ROWS_PER_WORKER = BATCH // NUM_WORKERS
REP = 8
BURSTS = ROWS_PER_WORKER // REP
K_SEM = 4
ROW_SC = MAX_LEN * D_MODEL

_mesh = plsc.VectorSubcoreMesh(core_axis_name="c", subcore_axis_name="s")


@functools.partial(
    pl.kernel,
    mesh=_mesh,
    out_type=jax.ShapeDtypeStruct((BATCH, ROW_SC), jnp.float32),
    scratch_types=[
        pltpu.VMEM((REP, ROW_SC), jnp.float32),
        pltpu.SemaphoreType.DMA((K_SEM,)),
    ],
)
def _broadcast_table(table_hbm, out_hbm, buf, sems):
    wid = lax.axis_index("s") * NUM_CORES + lax.axis_index("c")
    base = wid * ROWS_PER_WORKER
    for r in range(REP):
        pltpu.sync_copy(table_hbm, buf.at[pl.ds(r, 1)])

    copies = []
    for i in range(BURSTS):
        c = pltpu.make_async_copy(
            buf,
            out_hbm.at[pl.ds(base + i * REP, REP)],
            sems.at[i % K_SEM],
        )
        c.start()
        copies.append(c)
        if i >= K_SEM:
            copies[i - K_SEM].wait()
    for c in copies[max(0, BURSTS - K_SEM):]:
        c.wait()


ROW = MAX_LEN * D_MODEL
B_BLK = 256
N_CHUNKS = BATCH // B_BLK
K_INFLIGHT = 8


def _tc_body(table_ref, out_ref, scratch_ref, sem):
    scratch_ref[...] = jnp.broadcast_to(
        table_ref[...][None], (B_BLK, 2, ROW)
    )
    src = scratch_ref.at[:, pl.ds(0, 1)]
    copies = []
    for idx in range(N_CHUNKS):
        c = pltpu.make_async_copy(
            src,
            out_ref.at[pl.ds(idx * B_BLK, B_BLK)],
            sem.at[idx % K_INFLIGHT],
        )
        c.start()
        copies.append(c)
        if idx >= K_INFLIGHT:
            copies[idx - K_INFLIGHT].wait()
    for c in copies[max(0, N_CHUNKS - K_INFLIGHT):]:
        c.wait()


_tc_call = pl.pallas_call(
    _tc_body,
    in_specs=[pl.BlockSpec(memory_space=pltpu.VMEM)],
    out_specs=pl.BlockSpec(memory_space=pl.ANY),
    out_shape=jax.ShapeDtypeStruct((BATCH, 1, ROW), jnp.float32),
    scratch_shapes=[
        pltpu.VMEM((B_BLK, 2, ROW), jnp.float32),
        pltpu.SemaphoreType.DMA((K_INFLIGHT,)),
    ],
)


def kernel(x, pe_weight):
    del x
    flat = _tc_call(pe_weight.reshape(1, ROW))
    return flat.reshape(BATCH, MAX_LEN, D_MODEL)


# speedup vs baseline: 2.0845x; 2.0845x over previous
import functools

import jax
import jax.numpy as jnp
from jax import lax
from jax.experimental import pallas as pl
from jax.experimental.pallas import tpu as pltpu
from jax.experimental.pallas import tpu_sc as plsc

MAX_LEN = 200
D_MODEL = 64
BATCH = 4096

NUM_CORES = 2
NUM_SUBCORES = 16
NUM_WORKERS = NUM_CORES * NUM_SUBCORES
ROWS_PER_WORKER = BATCH // NUM_WORKERS
REP = 8
BURSTS = ROWS_PER_WORKER // REP
K_SEM = 4
ROW_SC = MAX_LEN * D_MODEL

_mesh = plsc.VectorSubcoreMesh(core_axis_name="c", subcore_axis_name="s")


@functools.partial(
    pl.kernel,
    mesh=_mesh,
    out_type=jax.ShapeDtypeStruct((BATCH, ROW_SC), jnp.float32),
    scratch_types=[
        pltpu.VMEM((REP, ROW_SC), jnp.float32),
        pltpu.SemaphoreType.DMA((K_SEM,)),
    ],
)
def _broadcast_table(table_hbm, out_hbm, buf, sems):
    wid = lax.axis_index("s") * NUM_CORES + lax.axis_index("c")
    base = wid * ROWS_PER_WORKER
    for r in range(REP):
        pltpu.sync_copy(table_hbm, buf.at[pl.ds(r, 1)])

    copies = []
    for i in range(BURSTS):
        c = pltpu.make_async_copy(
            buf,
            out_hbm.at[pl.ds(base + i * REP, REP)],
            sems.at[i % K_SEM],
        )
        c.start()
        copies.append(c)
        if i >= K_SEM:
            copies[i - K_SEM].wait()
    for c in copies[max(0, BURSTS - K_SEM):]:
        c.wait()


ROW = MAX_LEN * D_MODEL
B_BLK = 256
N_CHUNKS = BATCH // B_BLK
K_INFLIGHT = 8


def _tc_body(table_ref, out_ref, scratch_ref, sem):
    scratch_ref[...] = jnp.broadcast_to(
        table_ref[...][None], (B_BLK, 2, ROW)
    )
    src = scratch_ref.at[:, pl.ds(0, 1)]
    copies = []
    for idx in range(N_CHUNKS):
        c = pltpu.make_async_copy(
            src,
            out_ref.at[pl.ds(idx * B_BLK, B_BLK)],
            sem.at[idx % K_INFLIGHT],
        )
        c.start()
        copies.append(c)
        if idx >= K_INFLIGHT:
            copies[idx - K_INFLIGHT].wait()
    for c in copies[max(0, N_CHUNKS - K_INFLIGHT):]:
        c.wait()


_tc_call = pl.pallas_call(
    _tc_body,
    in_specs=[pl.BlockSpec(memory_space=pltpu.VMEM)],
    out_specs=pl.BlockSpec(memory_space=pl.ANY),
    out_shape=jax.ShapeDtypeStruct((BATCH, 1, ROW), jnp.float32),
    scratch_shapes=[
        pltpu.VMEM((B_BLK, 2, ROW), jnp.float32),
        pltpu.SemaphoreType.DMA((K_INFLIGHT,)),
    ],
)


B_TILE = 256
GRID = BATCH // B_TILE


def _tile_body(table_ref, out_ref):
    out_ref[...] = jnp.broadcast_to(table_ref[...], (B_TILE, ROW))


_tile_call = pl.pallas_call(
    _tile_body,
    grid=(GRID,),
    in_specs=[pl.BlockSpec((1, ROW), lambda i: (0, 0))],
    out_specs=pl.BlockSpec((B_TILE, ROW), lambda i: (i, 0)),
    out_shape=jax.ShapeDtypeStruct((BATCH, ROW), jnp.float32),
)


def kernel(x, pe_weight):
    del x
    flat = _tile_call(pe_weight.reshape(1, ROW))
    return flat.reshape(BATCH, MAX_LEN, D_MODEL)
